# baseline (speedup 1.0000x reference)
import functools

import jax
import jax.numpy as jnp
from jax.experimental import pallas as pl
from jax.experimental.pallas import tpu as pltpu

E = 8
TOP_K = 2
D = 2048
FF = 768

TT = 256


def _router_body(x_ref, gw_ref, logits_ref, combine_ref):
    x = x_ref[...]
    gw = gw_ref[...]
    logits = jax.lax.dot_general(
        x, gw, (((1,), (1,)), ((), ())), preferred_element_type=jnp.float32,
    )
    logits_ref[...] = logits
    m = jnp.max(logits, axis=1, keepdims=True)
    ex = jnp.exp(logits - m)
    probs = ex / jnp.sum(ex, axis=1, keepdims=True)
    iota = jax.lax.broadcasted_iota(jnp.int32, probs.shape, 1)
    p1 = jnp.max(probs, axis=1, keepdims=True)
    i1 = jnp.min(jnp.where(probs == p1, iota, E), axis=1, keepdims=True)
    masked = jnp.where(iota == i1, -1.0, probs)
    p2 = jnp.max(masked, axis=1, keepdims=True)
    i2 = jnp.min(jnp.where(masked == p2, iota, E), axis=1, keepdims=True)
    wsum = p1 + p2
    combine = jnp.where(iota == i1, p1, 0.0) + jnp.where(iota == i2, p2, 0.0)
    combine_ref[...] = combine / wsum


def _expert_body(x_ref, cmb_ref, wg_ref, wu_ref, wd_ref, out_ref):
    e = pl.program_id(1)
    x = x_ref[...]
    g = jax.lax.dot_general(
        x, wg_ref[0], (((1,), (1,)), ((), ())),
        preferred_element_type=jnp.float32,
    )
    u = jax.lax.dot_general(
        x, wu_ref[0], (((1,), (1,)), ((), ())),
        preferred_element_type=jnp.float32,
    )
    h = (g * jax.nn.sigmoid(g)) * u
    o = jax.lax.dot_general(
        h, wd_ref[0], (((1,), (1,)), ((), ())),
        preferred_element_type=jnp.float32,
    )
    c = cmb_ref[0, 0, :]
    term = o * c[:, None]

    @pl.when(e == 0)
    def _():
        out_ref[...] = term

    @pl.when(e > 0)
    def _():
        out_ref[...] += term


def kernel(hidden_states, gate_w, w_gate, w_up, w_down):
    B, S, _ = hidden_states.shape
    T = B * S
    x = hidden_states.reshape(T, D)

    logits, combine = pl.pallas_call(
        _router_body,
        grid=(1,),
        in_specs=[
            pl.BlockSpec((T, D), lambda i: (0, 0)),
            pl.BlockSpec((E, D), lambda i: (0, 0)),
        ],
        out_specs=[
            pl.BlockSpec((T, E), lambda i: (0, 0)),
            pl.BlockSpec((T, E), lambda i: (0, 0)),
        ],
        out_shape=[
            jax.ShapeDtypeStruct((T, E), jnp.float32),
            jax.ShapeDtypeStruct((T, E), jnp.float32),
        ],
    )(x, gate_w)

    combine_t = combine.T.reshape(E, 1, T)

    out = pl.pallas_call(
        _expert_body,
        grid=(T // TT, E),
        in_specs=[
            pl.BlockSpec((TT, D), lambda i, e: (i, 0)),
            pl.BlockSpec((1, 1, TT), lambda i, e: (e, 0, i)),
            pl.BlockSpec((1, FF, D), lambda i, e: (e, 0, 0)),
            pl.BlockSpec((1, FF, D), lambda i, e: (e, 0, 0)),
            pl.BlockSpec((1, D, FF), lambda i, e: (e, 0, 0)),
        ],
        out_specs=pl.BlockSpec((TT, D), lambda i, e: (i, 0)),
        out_shape=jax.ShapeDtypeStruct((T, D), jnp.float32),
        compiler_params=pltpu.CompilerParams(
            dimension_semantics=("arbitrary", "arbitrary"),
        ),
    )(x, combine_t, w_gate, w_up, w_down)

    return out.reshape(B, S, D), logits


# speedup vs baseline: 1.3169x; 1.3169x over previous
import functools

import jax
import jax.numpy as jnp
from jax import lax
from jax.experimental import pallas as pl
from jax.experimental.pallas import tpu as pltpu
from jax.experimental.pallas import tpu_sc as plsc

E = 8
D = 2048
FF = 768
T = 2048
K = 2
TTG = 128
P = T * K + E * TTG
G = P // TTG

NW = 32
CH = 16


def _router_body(x_ref, gw_ref, logits_ref, pos_ref, w2_ref, tmap_ref):
    x = x_ref[...]
    gw = gw_ref[...]
    logits = jax.lax.dot_general(
        x, gw, (((1,), (1,)), ((), ())), preferred_element_type=jnp.float32,
    )
    logits_ref[...] = logits
    m = jnp.max(logits, axis=1, keepdims=True)
    ex = jnp.exp(logits - m)
    probs = ex / jnp.sum(ex, axis=1, keepdims=True)
    iota = jax.lax.broadcasted_iota(jnp.int32, probs.shape, 1)
    p1 = jnp.max(probs, axis=1, keepdims=True)
    i1 = jnp.min(jnp.where(probs == p1, iota, E), axis=1, keepdims=True)
    masked = jnp.where(iota == i1, -1.0, probs)
    p2 = jnp.max(masked, axis=1, keepdims=True)
    i2 = jnp.min(jnp.where(masked == p2, iota, E), axis=1, keepdims=True)
    wsum = p1 + p2
    w2_ref[0, :] = (p1 / wsum)[:, 0]
    w2_ref[1, :] = (p2 / wsum)[:, 0]

    oh1 = (iota == i1).astype(jnp.float32)
    oh2 = (iota == i2).astype(jnp.float32)
    cnt1 = jnp.sum(oh1, axis=0, keepdims=True)
    cnt2 = jnp.sum(oh2, axis=0, keepdims=True)
    cnt = cnt1 + cnt2
    padded = jnp.floor((cnt + (TTG - 1.0)) * (1.0 / TTG)) * TTG
    r8 = jax.lax.broadcasted_iota(jnp.int32, (E, E), 0)
    c8 = jax.lax.broadcasted_iota(jnp.int32, (E, E), 1)
    u8 = (r8 < c8).astype(jnp.float32)
    start = jax.lax.dot_general(
        padded, u8, (((1,), (0,)), ((), ())),
        preferred_element_type=jnp.float32,
        precision=jax.lax.Precision.HIGHEST,
    )
    off2 = start + cnt1

    CT = 512
    rl = jax.lax.broadcasted_iota(jnp.int32, (CT, CT), 0)
    cl = jax.lax.broadcasted_iota(jnp.int32, (CT, CT), 1)
    ltri = (cl < rl).astype(jnp.float32)
    run1 = jnp.zeros((1, E), jnp.float32)
    run2 = jnp.zeros((1, E), jnp.float32)
    for c in range(T // CT):
        sl = slice(c * CT, (c + 1) * CT)
        o1 = oh1[sl]
        e1 = jax.lax.dot_general(
            ltri, o1, (((1,), (0,)), ((), ())),
            preferred_element_type=jnp.float32,
            precision=jax.lax.Precision.HIGHEST,
        ) + run1
        pos_ref[0, sl] = jnp.sum(o1 * (start + e1), axis=1).astype(jnp.int32)
        run1 = run1 + jnp.sum(o1, axis=0, keepdims=True)
        o2 = oh2[sl]
        e2 = jax.lax.dot_general(
            ltri, o2, (((1,), (0,)), ((), ())),
            preferred_element_type=jnp.float32,
            precision=jax.lax.Precision.HIGHEST,
        ) + run2
        pos_ref[1, sl] = jnp.sum(o2 * (off2 + e2), axis=1).astype(jnp.int32)
        run2 = run2 + jnp.sum(o2, axis=0, keepdims=True)

    gi = jax.lax.broadcasted_iota(jnp.int32, (128, E), 0).astype(jnp.float32) * TTG
    mape = jnp.sum((gi >= start).astype(jnp.int32), axis=1) - 1
    used = jnp.sum(padded)
    mape = jnp.where(gi[:, 0] < used, mape, -1)
    tmap_ref[0, :] = mape


def _dispatch_body(x_hbm, posr_hbm, pos16_hbm, war_hbm, xs_hbm, aw_hbm,
                   xb0, xb1, posb, posb16, wb, wr0, wr1, gsem, s0, s1, aw0, aw1):
    wid = lax.axis_index("s") * 2 + lax.axis_index("c")
    k = wid // 16
    tb = wid % 16
    t0 = tb * 128

    pltpu.sync_copy(posr_hbm.at[k, tb], posb)
    pltpu.sync_copy(pos16_hbm.at[k, tb], posb16)
    pltpu.sync_copy(war_hbm.at[k, tb], wb)

    xbufs = (xb0, xb1)
    ssems = (s0, s1)
    scat = [None, None]
    for j in range(128 // CH):
        b = xbufs[j % 2]
        if scat[j % 2] is not None:
            scat[j % 2].wait()
        pltpu.sync_copy(x_hbm.at[pl.ds(t0 + CH * j, CH)], b)
        scat[j % 2] = pltpu.async_copy(b, xs_hbm.at[posb.at[j]], ssems[j % 2])
    scat[0].wait()
    scat[1].wait()

    wrows = (wr0, wr1)
    awsems = (aw0, aw1)
    wscat = [None, None]
    for j in range(8):
        jv = jnp.full((16,), j, jnp.int32)
        wr = wrows[j % 2]
        if wscat[j % 2] is not None:
            wscat[j % 2].wait()
        for l in range(16):
            lv = jnp.full((16,), l, jnp.int32)
            wr[l, 0:16] = plsc.load_gather(wb, [jv, lv])
        wscat[j % 2] = pltpu.async_copy(
            wr, aw_hbm.at[posb16.at[j]], awsems[j % 2])
    wscat[0].wait()
    wscat[1].wait()


def _expert_body(m_ref, xs_ref, aw_ref, wg_ref, wu_ref, wd_ref, out_ref):
    g = pl.program_id(0)
    e = m_ref[g]

    @pl.when(e >= 0)
    def _():
        x = xs_ref[...]
        gg = jax.lax.dot_general(
            x, wg_ref[0], (((1,), (1,)), ((), ())),
            preferred_element_type=jnp.float32,
        )
        uu = jax.lax.dot_general(
            x, wu_ref[0], (((1,), (1,)), ((), ())),
            preferred_element_type=jnp.float32,
        )
        h = (gg * jax.nn.sigmoid(gg)) * uu
        o = jax.lax.dot_general(
            h.astype(xs_ref.dtype), wd_ref[0], (((1,), (1,)), ((), ())),
            preferred_element_type=jnp.float32,
        )
        out_ref[...] = o * aw_ref[:, 0:1]


def _combine_body(os_hbm, pos16_hbm, out_hbm, b0, b1, posb, g0, g1):
    wid = lax.axis_index("s") * 2 + lax.axis_index("c")
    k = wid // 16
    tb = wid % 16

    pltpu.sync_copy(pos16_hbm.at[k, tb], posb)
    bufs = (b0, b1)
    sems = (g0, g1)
    outd = [None, None]
    for j in range(8):
        b = bufs[j % 2]
        if outd[j % 2] is not None:
            outd[j % 2].wait()
        pltpu.sync_copy(os_hbm.at[posb.at[j]], b)
        outd[j % 2] = pltpu.async_copy(
            b, out_hbm.at[pl.ds(k * T + tb * 128 + 16 * j, 16)], sems[j % 2])
    outd[0].wait()
    outd[1].wait()


def _add_body(a_ref, b_ref, o_ref):
    o_ref[...] = a_ref[...] + b_ref[...]


def kernel(hidden_states, gate_w, w_gate, w_up, w_down):
    B, S, _ = hidden_states.shape
    x = hidden_states.reshape(T, D)

    logits, pos2, w2, tmap = pl.pallas_call(
        _router_body,
        grid=(1,),
        in_specs=[
            pl.BlockSpec((T, D), lambda i: (0, 0)),
            pl.BlockSpec((E, D), lambda i: (0, 0)),
        ],
        out_specs=[
            pl.BlockSpec((T, E), lambda i: (0, 0)),
            pl.BlockSpec((K, T), lambda i: (0, 0)),
            pl.BlockSpec((K, T), lambda i: (0, 0)),
            pl.BlockSpec((1, 128), lambda i: (0, 0)),
        ],
        out_shape=[
            jax.ShapeDtypeStruct((T, E), jnp.float32),
            jax.ShapeDtypeStruct((K, T), jnp.int32),
            jax.ShapeDtypeStruct((K, T), jnp.float32),
            jax.ShapeDtypeStruct((1, 128), jnp.int32),
        ],
    )(x, gate_w)

    posr = pos2.reshape(K, 16, 128 // CH, CH)
    pos16 = pos2.reshape(K, 16, 8, 16)
    war = w2.reshape(K, 16, 8, 16)

    mesh = plsc.VectorSubcoreMesh(core_axis_name="c", subcore_axis_name="s")
    xs, aw = pl.kernel(
        _dispatch_body,
        out_type=[
            jax.ShapeDtypeStruct((P, D), jnp.float32),
            jax.ShapeDtypeStruct((P, 128), jnp.float32),
        ],
        mesh=mesh,
        compiler_params=pltpu.CompilerParams(needs_layout_passes=False),
        scratch_types=[
            pltpu.VMEM((CH, D), jnp.float32),
            pltpu.VMEM((CH, D), jnp.float32),
            pltpu.VMEM((128 // CH, CH), jnp.int32),
            pltpu.VMEM((8, 16), jnp.int32),
            pltpu.VMEM((8, 16), jnp.float32),
            pltpu.VMEM((16, 128), jnp.float32),
            pltpu.VMEM((16, 128), jnp.float32),
            pltpu.SemaphoreType.DMA,
            pltpu.SemaphoreType.DMA,
            pltpu.SemaphoreType.DMA,
            pltpu.SemaphoreType.DMA,
            pltpu.SemaphoreType.DMA,
        ],
    )(x, posr, pos16, war)

    grid_spec = pltpu.PrefetchScalarGridSpec(
        num_scalar_prefetch=1,
        grid=(G,),
        in_specs=[
            pl.BlockSpec((TTG, D), lambda g, m: (g, 0)),
            pl.BlockSpec((TTG, 128), lambda g, m: (g, 0)),
            pl.BlockSpec((1, FF, D), lambda g, m: (jnp.maximum(m[g], 0), 0, 0)),
            pl.BlockSpec((1, FF, D), lambda g, m: (jnp.maximum(m[g], 0), 0, 0)),
            pl.BlockSpec((1, D, FF), lambda g, m: (jnp.maximum(m[g], 0), 0, 0)),
        ],
        out_specs=pl.BlockSpec((TTG, D), lambda g, m: (g, 0)),
    )
    os = pl.pallas_call(
        _expert_body,
        grid_spec=grid_spec,
        out_shape=jax.ShapeDtypeStruct((P, D), jnp.float32),
        compiler_params=pltpu.CompilerParams(
            dimension_semantics=("arbitrary",),
        ),
    )(tmap[0], xs, aw, w_gate, w_up, w_down)

    out2 = pl.kernel(
        _combine_body,
        out_type=jax.ShapeDtypeStruct((K * T, D), jnp.float32),
        mesh=plsc.VectorSubcoreMesh(core_axis_name="c", subcore_axis_name="s"),
        compiler_params=pltpu.CompilerParams(needs_layout_passes=False),
        scratch_types=[
            pltpu.VMEM((16, D), jnp.float32),
            pltpu.VMEM((16, D), jnp.float32),
            pltpu.VMEM((8, 16), jnp.int32),
            pltpu.SemaphoreType.DMA,
            pltpu.SemaphoreType.DMA,
        ],
    )(os, pos16)

    out = pl.pallas_call(
        _add_body,
        grid=(16,),
        in_specs=[
            pl.BlockSpec((128, D), lambda i: (i, 0)),
            pl.BlockSpec((128, D), lambda i: (i + 16, 0)),
        ],
        out_specs=pl.BlockSpec((128, D), lambda i: (i, 0)),
        out_shape=jax.ShapeDtypeStruct((T, D), jnp.float32),
    )(out2, out2)

    return out.reshape(B, S, D), logits


# speedup vs baseline: 1.3232x; 1.0048x over previous
import functools

import jax
import jax.numpy as jnp
from jax import lax
from jax.experimental import pallas as pl
from jax.experimental.pallas import tpu as pltpu
from jax.experimental.pallas import tpu_sc as plsc

E = 8
D = 2048
FF = 768
T = 2048
K = 2
TTG = 128
P = T * K + E * TTG
G = P // TTG

NW = 32
CH = 16


def _router_body(x_ref, gw_ref, logits_ref, pos_ref, w2_ref, tmap_ref):
    x = x_ref[...]
    gw = gw_ref[...]
    logits = jax.lax.dot_general(
        x, gw, (((1,), (1,)), ((), ())), preferred_element_type=jnp.float32,
    )
    logits_ref[...] = logits
    m = jnp.max(logits, axis=1, keepdims=True)
    ex = jnp.exp(logits - m)
    probs = ex / jnp.sum(ex, axis=1, keepdims=True)
    iota = jax.lax.broadcasted_iota(jnp.int32, probs.shape, 1)
    p1 = jnp.max(probs, axis=1, keepdims=True)
    i1 = jnp.min(jnp.where(probs == p1, iota, E), axis=1, keepdims=True)
    masked = jnp.where(iota == i1, -1.0, probs)
    p2 = jnp.max(masked, axis=1, keepdims=True)
    i2 = jnp.min(jnp.where(masked == p2, iota, E), axis=1, keepdims=True)
    wsum = p1 + p2
    w2_ref[:, 0:1] = p1 / wsum
    w2_ref[:, 1:2] = p2 / wsum

    oh1 = (iota == i1).astype(jnp.float32)
    oh2 = (iota == i2).astype(jnp.float32)
    cnt1 = jnp.sum(oh1, axis=0, keepdims=True)
    cnt2 = jnp.sum(oh2, axis=0, keepdims=True)
    cnt = cnt1 + cnt2
    padded = jnp.floor((cnt + (TTG - 1.0)) * (1.0 / TTG)) * TTG
    r8 = jax.lax.broadcasted_iota(jnp.int32, (E, E), 0)
    c8 = jax.lax.broadcasted_iota(jnp.int32, (E, E), 1)
    u8 = (r8 < c8).astype(jnp.float32)
    start = jax.lax.dot_general(
        padded, u8, (((1,), (0,)), ((), ())),
        preferred_element_type=jnp.float32,
        precision=jax.lax.Precision.HIGHEST,
    )
    off2 = start + cnt1

    CT = 512
    rl = jax.lax.broadcasted_iota(jnp.int32, (CT, CT), 0)
    cl = jax.lax.broadcasted_iota(jnp.int32, (CT, CT), 1)
    ltri = (cl < rl).astype(jnp.float32)
    run1 = jnp.zeros((1, E), jnp.float32)
    run2 = jnp.zeros((1, E), jnp.float32)
    for c in range(T // CT):
        sl = slice(c * CT, (c + 1) * CT)
        o1 = oh1[sl]
        e1 = jax.lax.dot_general(
            ltri, o1, (((1,), (0,)), ((), ())),
            preferred_element_type=jnp.float32,
            precision=jax.lax.Precision.HIGHEST,
        ) + run1
        pos_ref[0, sl] = jnp.sum(o1 * (start + e1), axis=1).astype(jnp.int32)
        run1 = run1 + jnp.sum(o1, axis=0, keepdims=True)
        o2 = oh2[sl]
        e2 = jax.lax.dot_general(
            ltri, o2, (((1,), (0,)), ((), ())),
            preferred_element_type=jnp.float32,
            precision=jax.lax.Precision.HIGHEST,
        ) + run2
        pos_ref[1, sl] = jnp.sum(o2 * (off2 + e2), axis=1).astype(jnp.int32)
        run2 = run2 + jnp.sum(o2, axis=0, keepdims=True)

    gi = jax.lax.broadcasted_iota(jnp.int32, (128, E), 0).astype(jnp.float32) * TTG
    mape = jnp.sum((gi >= start).astype(jnp.int32), axis=1) - 1
    used = jnp.sum(padded)
    mape = jnp.where(gi[:, 0] < used, mape, -1)
    tmap_ref[0, :] = mape


def _dispatch_body(x_hbm, posr_hbm, xs_hbm, xb0, xb1, posb, gsem, s0, s1):
    wid = lax.axis_index("s") * 2 + lax.axis_index("c")
    k = wid // 16
    tb = wid % 16
    t0 = tb * 128

    pltpu.sync_copy(posr_hbm.at[k, tb], posb)

    xbufs = (xb0, xb1)
    ssems = (s0, s1)
    scat = [None, None]
    for j in range(128 // CH):
        b = xbufs[j % 2]
        if scat[j % 2] is not None:
            scat[j % 2].wait()
        pltpu.sync_copy(x_hbm.at[pl.ds(t0 + CH * j, CH)], b)
        scat[j % 2] = pltpu.async_copy(b, xs_hbm.at[posb.at[j]], ssems[j % 2])
    scat[0].wait()
    scat[1].wait()


def _expert_body(m_ref, xs_ref, wg_ref, wu_ref, wd_ref, out_ref):
    g = pl.program_id(0)
    e = m_ref[g]

    @pl.when(e >= 0)
    def _():
        x = xs_ref[...].astype(jnp.bfloat16)
        gg = jax.lax.dot_general(
            x, wg_ref[0].astype(jnp.bfloat16), (((1,), (1,)), ((), ())),
            preferred_element_type=jnp.float32,
        )
        uu = jax.lax.dot_general(
            x, wu_ref[0].astype(jnp.bfloat16), (((1,), (1,)), ((), ())),
            preferred_element_type=jnp.float32,
        )
        h = (gg * jax.nn.sigmoid(gg)) * uu
        o = jax.lax.dot_general(
            h.astype(jnp.bfloat16), wd_ref[0].astype(jnp.bfloat16),
            (((1,), (1,)), ((), ())),
            preferred_element_type=jnp.float32,
        )
        out_ref[...] = o


def _combine_body(os_hbm, pos16_hbm, out_hbm, b0, b1, posb, g0, g1):
    wid = lax.axis_index("s") * 2 + lax.axis_index("c")
    k = wid // 16
    tb = wid % 16

    pltpu.sync_copy(pos16_hbm.at[k, tb], posb)
    bufs = (b0, b1)
    sems = (g0, g1)
    outd = [None, None]
    for j in range(8):
        b = bufs[j % 2]
        if outd[j % 2] is not None:
            outd[j % 2].wait()
        pltpu.sync_copy(os_hbm.at[posb.at[j]], b)
        outd[j % 2] = pltpu.async_copy(
            b, out_hbm.at[pl.ds(k * T + tb * 128 + 16 * j, 16)], sems[j % 2])
    outd[0].wait()
    outd[1].wait()


def _wadd_body(a_ref, b_ref, w_ref, o_ref):
    a = a_ref[...].astype(jnp.float32)
    b = b_ref[...].astype(jnp.float32)
    o_ref[...] = a * w_ref[:, 0:1] + b * w_ref[:, 1:2]


def kernel(hidden_states, gate_w, w_gate, w_up, w_down):
    B, S, _ = hidden_states.shape
    x = hidden_states.reshape(T, D)

    logits, pos2, w2, tmap = pl.pallas_call(
        _router_body,
        grid=(1,),
        in_specs=[
            pl.BlockSpec((T, D), lambda i: (0, 0)),
            pl.BlockSpec((E, D), lambda i: (0, 0)),
        ],
        out_specs=[
            pl.BlockSpec((T, E), lambda i: (0, 0)),
            pl.BlockSpec((K, T), lambda i: (0, 0)),
            pl.BlockSpec((T, K), lambda i: (0, 0)),
            pl.BlockSpec((1, 128), lambda i: (0, 0)),
        ],
        out_shape=[
            jax.ShapeDtypeStruct((T, E), jnp.float32),
            jax.ShapeDtypeStruct((K, T), jnp.int32),
            jax.ShapeDtypeStruct((T, K), jnp.float32),
            jax.ShapeDtypeStruct((1, 128), jnp.int32),
        ],
    )(x, gate_w)

    posr = pos2.reshape(K, 16, 128 // CH, CH)
    pos16 = pos2.reshape(K, 16, 8, 16)

    mesh = plsc.VectorSubcoreMesh(core_axis_name="c", subcore_axis_name="s")
    xs = pl.kernel(
        _dispatch_body,
        out_type=jax.ShapeDtypeStruct((P, D), jnp.float32),
        mesh=mesh,
        compiler_params=pltpu.CompilerParams(needs_layout_passes=False),
        scratch_types=[
            pltpu.VMEM((CH, D), jnp.float32),
            pltpu.VMEM((CH, D), jnp.float32),
            pltpu.VMEM((128 // CH, CH), jnp.int32),
            pltpu.SemaphoreType.DMA,
            pltpu.SemaphoreType.DMA,
            pltpu.SemaphoreType.DMA,
        ],
    )(x, posr)

    grid_spec = pltpu.PrefetchScalarGridSpec(
        num_scalar_prefetch=1,
        grid=(G,),
        in_specs=[
            pl.BlockSpec((TTG, D), lambda g, m: (g, 0)),
            pl.BlockSpec((1, FF, D), lambda g, m: (jnp.maximum(m[g], 0), 0, 0)),
            pl.BlockSpec((1, FF, D), lambda g, m: (jnp.maximum(m[g], 0), 0, 0)),
            pl.BlockSpec((1, D, FF), lambda g, m: (jnp.maximum(m[g], 0), 0, 0)),
        ],
        out_specs=pl.BlockSpec((TTG, D), lambda g, m: (g, 0)),
    )
    os = pl.pallas_call(
        _expert_body,
        grid_spec=grid_spec,
        out_shape=jax.ShapeDtypeStruct((P, D), jnp.float32),
        compiler_params=pltpu.CompilerParams(
            dimension_semantics=("arbitrary",),
        ),
    )(tmap[0], xs, w_gate, w_up, w_down)

    out2 = pl.kernel(
        _combine_body,
        out_type=jax.ShapeDtypeStruct((K * T, D), jnp.float32),
        mesh=plsc.VectorSubcoreMesh(core_axis_name="c", subcore_axis_name="s"),
        compiler_params=pltpu.CompilerParams(needs_layout_passes=False),
        scratch_types=[
            pltpu.VMEM((16, D), jnp.float32),
            pltpu.VMEM((16, D), jnp.float32),
            pltpu.VMEM((8, 16), jnp.int32),
            pltpu.SemaphoreType.DMA,
            pltpu.SemaphoreType.DMA,
        ],
    )(os, pos16)

    out = pl.pallas_call(
        _wadd_body,
        grid=(16,),
        in_specs=[
            pl.BlockSpec((128, D), lambda i: (i, 0)),
            pl.BlockSpec((128, D), lambda i: (i + 16, 0)),
            pl.BlockSpec((128, K), lambda i: (i, 0)),
        ],
        out_specs=pl.BlockSpec((128, D), lambda i: (i, 0)),
        out_shape=jax.ShapeDtypeStruct((T, D), jnp.float32),
    )(out2, out2, w2)

    return out.reshape(B, S, D), logits


# speedup vs baseline: 1.6349x; 1.2356x over previous
import functools

import jax
import jax.numpy as jnp
from jax import lax
from jax.experimental import pallas as pl
from jax.experimental.pallas import tpu as pltpu
from jax.experimental.pallas import tpu_sc as plsc

E = 8
D = 2048
FF = 768
T = 2048
K = 2
TTG = 256
P = T * K + E * TTG
G = P // TTG

NW = 32
CH = 16


def _router_body(x_ref, gw_ref, logits_ref, pos_ref, w2_ref, tmap_ref):
    x = x_ref[...]
    gw = gw_ref[...]
    logits = jax.lax.dot_general(
        x, gw, (((1,), (1,)), ((), ())), preferred_element_type=jnp.float32,
    )
    logits_ref[...] = logits
    m = jnp.max(logits, axis=1, keepdims=True)
    ex = jnp.exp(logits - m)
    probs = ex / jnp.sum(ex, axis=1, keepdims=True)
    iota = jax.lax.broadcasted_iota(jnp.int32, probs.shape, 1)
    p1 = jnp.max(probs, axis=1, keepdims=True)
    i1 = jnp.min(jnp.where(probs == p1, iota, E), axis=1, keepdims=True)
    masked = jnp.where(iota == i1, -1.0, probs)
    p2 = jnp.max(masked, axis=1, keepdims=True)
    i2 = jnp.min(jnp.where(masked == p2, iota, E), axis=1, keepdims=True)
    wsum = p1 + p2
    w2_ref[:, 0:1] = p1 / wsum
    w2_ref[:, 1:2] = p2 / wsum

    oh1 = (iota == i1).astype(jnp.float32)
    oh2 = (iota == i2).astype(jnp.float32)
    cnt1 = jnp.sum(oh1, axis=0, keepdims=True)
    cnt2 = jnp.sum(oh2, axis=0, keepdims=True)
    cnt = cnt1 + cnt2
    padded = jnp.floor((cnt + (TTG - 1.0)) * (1.0 / TTG)) * TTG
    r8 = jax.lax.broadcasted_iota(jnp.int32, (E, E), 0)
    c8 = jax.lax.broadcasted_iota(jnp.int32, (E, E), 1)
    u8 = (r8 < c8).astype(jnp.float32)
    start = jax.lax.dot_general(
        padded, u8, (((1,), (0,)), ((), ())),
        preferred_element_type=jnp.float32,
        precision=jax.lax.Precision.HIGHEST,
    )
    off2 = start + cnt1

    CT = 512
    rl = jax.lax.broadcasted_iota(jnp.int32, (CT, CT), 0)
    cl = jax.lax.broadcasted_iota(jnp.int32, (CT, CT), 1)
    ltri = (cl < rl).astype(jnp.float32)
    run1 = jnp.zeros((1, E), jnp.float32)
    run2 = jnp.zeros((1, E), jnp.float32)
    for c in range(T // CT):
        sl = slice(c * CT, (c + 1) * CT)
        o1 = oh1[sl]
        e1 = jax.lax.dot_general(
            ltri, o1, (((1,), (0,)), ((), ())),
            preferred_element_type=jnp.float32,
            precision=jax.lax.Precision.HIGHEST,
        ) + run1
        pos_ref[0, sl] = jnp.sum(o1 * (start + e1), axis=1).astype(jnp.int32)
        run1 = run1 + jnp.sum(o1, axis=0, keepdims=True)
        o2 = oh2[sl]
        e2 = jax.lax.dot_general(
            ltri, o2, (((1,), (0,)), ((), ())),
            preferred_element_type=jnp.float32,
            precision=jax.lax.Precision.HIGHEST,
        ) + run2
        pos_ref[1, sl] = jnp.sum(o2 * (off2 + e2), axis=1).astype(jnp.int32)
        run2 = run2 + jnp.sum(o2, axis=0, keepdims=True)

    gi = jax.lax.broadcasted_iota(jnp.int32, (128, E), 0).astype(jnp.float32) * TTG
    mape = jnp.sum((gi >= start).astype(jnp.int32), axis=1) - 1
    used = jnp.sum(padded)
    mape = jnp.where(gi[:, 0] < used, mape, -1)
    tmap_ref[0, :] = mape


def _dispatch_body(x_hbm, posr_hbm, xs_hbm, xb0, xb1, posb, gsem, s0, s1):
    wid = lax.axis_index("s") * 2 + lax.axis_index("c")
    k = wid // 16
    tb = wid % 16
    t0 = tb * 128

    pltpu.sync_copy(posr_hbm.at[k, tb], posb)

    xbufs = (xb0, xb1)
    ssems = (s0, s1)
    scat = [None, None]
    for j in range(128 // CH):
        b = xbufs[j % 2]
        if scat[j % 2] is not None:
            scat[j % 2].wait()
        pltpu.sync_copy(x_hbm.at[pl.ds(t0 + CH * j, CH)], b)
        scat[j % 2] = pltpu.async_copy(b, xs_hbm.at[posb.at[j]], ssems[j % 2])
    scat[0].wait()
    scat[1].wait()


def _expert_body(m_ref, xs_ref, wg_ref, wu_ref, wd_ref, out_ref):
    g = pl.program_id(0)
    e = m_ref[g]

    @pl.when(e >= 0)
    def _():
        x = xs_ref[...].astype(jnp.bfloat16)
        gg = jax.lax.dot_general(
            x, wg_ref[0].astype(jnp.bfloat16), (((1,), (1,)), ((), ())),
            preferred_element_type=jnp.float32,
        )
        uu = jax.lax.dot_general(
            x, wu_ref[0].astype(jnp.bfloat16), (((1,), (1,)), ((), ())),
            preferred_element_type=jnp.float32,
        )
        h = (gg * jax.nn.sigmoid(gg)) * uu
        o = jax.lax.dot_general(
            h.astype(jnp.bfloat16), wd_ref[0].astype(jnp.bfloat16),
            (((1,), (1,)), ((), ())),
            preferred_element_type=jnp.float32,
        )
        out_ref[...] = o


def _combine_body(os_hbm, pos16_hbm, out_hbm, b0, b1, posb, g0, g1):
    wid = lax.axis_index("s") * 2 + lax.axis_index("c")
    k = wid // 16
    tb = wid % 16

    pltpu.sync_copy(pos16_hbm.at[k, tb], posb)
    bufs = (b0, b1)
    sems = (g0, g1)
    outd = [None, None]
    for j in range(8):
        b = bufs[j % 2]
        if outd[j % 2] is not None:
            outd[j % 2].wait()
        pltpu.sync_copy(os_hbm.at[posb.at[j]], b)
        outd[j % 2] = pltpu.async_copy(
            b, out_hbm.at[pl.ds(k * T + tb * 128 + 16 * j, 16)], sems[j % 2])
    outd[0].wait()
    outd[1].wait()


def _wadd_body(a_ref, b_ref, w_ref, o_ref):
    a = a_ref[...].astype(jnp.float32)
    b = b_ref[...].astype(jnp.float32)
    o_ref[...] = a * w_ref[:, 0:1] + b * w_ref[:, 1:2]


def kernel(hidden_states, gate_w, w_gate, w_up, w_down):
    B, S, _ = hidden_states.shape
    x = hidden_states.reshape(T, D)

    logits, pos2, w2, tmap = pl.pallas_call(
        _router_body,
        grid=(1,),
        in_specs=[
            pl.BlockSpec((T, D), lambda i: (0, 0)),
            pl.BlockSpec((E, D), lambda i: (0, 0)),
        ],
        out_specs=[
            pl.BlockSpec((T, E), lambda i: (0, 0)),
            pl.BlockSpec((K, T), lambda i: (0, 0)),
            pl.BlockSpec((T, K), lambda i: (0, 0)),
            pl.BlockSpec((1, 128), lambda i: (0, 0)),
        ],
        out_shape=[
            jax.ShapeDtypeStruct((T, E), jnp.float32),
            jax.ShapeDtypeStruct((K, T), jnp.int32),
            jax.ShapeDtypeStruct((T, K), jnp.float32),
            jax.ShapeDtypeStruct((1, 128), jnp.int32),
        ],
    )(x, gate_w)

    posr = pos2.reshape(K, 16, 128 // CH, CH)
    pos16 = pos2.reshape(K, 16, 8, 16)

    mesh = plsc.VectorSubcoreMesh(core_axis_name="c", subcore_axis_name="s")
    xs = pl.kernel(
        _dispatch_body,
        out_type=jax.ShapeDtypeStruct((P, D), jnp.float32),
        mesh=mesh,
        compiler_params=pltpu.CompilerParams(needs_layout_passes=False),
        scratch_types=[
            pltpu.VMEM((CH, D), jnp.float32),
            pltpu.VMEM((CH, D), jnp.float32),
            pltpu.VMEM((128 // CH, CH), jnp.int32),
            pltpu.SemaphoreType.DMA,
            pltpu.SemaphoreType.DMA,
            pltpu.SemaphoreType.DMA,
        ],
    )(x, posr)

    grid_spec = pltpu.PrefetchScalarGridSpec(
        num_scalar_prefetch=1,
        grid=(G,),
        in_specs=[
            pl.BlockSpec((TTG, D), lambda g, m: (g, 0)),
            pl.BlockSpec((1, FF, D), lambda g, m: (jnp.maximum(m[g], 0), 0, 0)),
            pl.BlockSpec((1, FF, D), lambda g, m: (jnp.maximum(m[g], 0), 0, 0)),
            pl.BlockSpec((1, D, FF), lambda g, m: (jnp.maximum(m[g], 0), 0, 0)),
        ],
        out_specs=pl.BlockSpec((TTG, D), lambda g, m: (g, 0)),
    )
    os = pl.pallas_call(
        _expert_body,
        grid_spec=grid_spec,
        out_shape=jax.ShapeDtypeStruct((P, D), jnp.float32),
        compiler_params=pltpu.CompilerParams(
            dimension_semantics=("arbitrary",),
        ),
    )(tmap[0], xs, w_gate, w_up, w_down)

    out2 = pl.kernel(
        _combine_body,
        out_type=jax.ShapeDtypeStruct((K * T, D), jnp.float32),
        mesh=plsc.VectorSubcoreMesh(core_axis_name="c", subcore_axis_name="s"),
        compiler_params=pltpu.CompilerParams(needs_layout_passes=False),
        scratch_types=[
            pltpu.VMEM((16, D), jnp.float32),
            pltpu.VMEM((16, D), jnp.float32),
            pltpu.VMEM((8, 16), jnp.int32),
            pltpu.SemaphoreType.DMA,
            pltpu.SemaphoreType.DMA,
        ],
    )(os, pos16)

    out = pl.pallas_call(
        _wadd_body,
        grid=(16,),
        in_specs=[
            pl.BlockSpec((128, D), lambda i: (i, 0)),
            pl.BlockSpec((128, D), lambda i: (i + 16, 0)),
            pl.BlockSpec((128, K), lambda i: (i, 0)),
        ],
        out_specs=pl.BlockSpec((128, D), lambda i: (i, 0)),
        out_shape=jax.ShapeDtypeStruct((T, D), jnp.float32),
    )(out2, out2, w2)

    return out.reshape(B, S, D), logits


# speedup vs baseline: 1.7202x; 1.0522x over previous
import functools

import jax
import jax.numpy as jnp
from jax import lax
from jax.experimental import pallas as pl
from jax.experimental.pallas import tpu as pltpu
from jax.experimental.pallas import tpu_sc as plsc

E = 8
D = 2048
FF = 768
T = 2048
K = 2
TTG = 256
P = T * K + E * TTG
G = P // TTG

NW = 32
CH = 16


def _router_body(x_ref, gw_ref, logits_ref, pos_ref, w2_ref, tmap_ref):
    x = x_ref[...]
    gw = gw_ref[...]
    logits = jax.lax.dot_general(
        x, gw, (((1,), (1,)), ((), ())), preferred_element_type=jnp.float32,
    )
    logits_ref[...] = logits
    m = jnp.max(logits, axis=1, keepdims=True)
    ex = jnp.exp(logits - m)
    probs = ex / jnp.sum(ex, axis=1, keepdims=True)
    iota = jax.lax.broadcasted_iota(jnp.int32, probs.shape, 1)
    p1 = jnp.max(probs, axis=1, keepdims=True)
    i1 = jnp.min(jnp.where(probs == p1, iota, E), axis=1, keepdims=True)
    masked = jnp.where(iota == i1, -1.0, probs)
    p2 = jnp.max(masked, axis=1, keepdims=True)
    i2 = jnp.min(jnp.where(masked == p2, iota, E), axis=1, keepdims=True)
    wsum = p1 + p2
    w2_ref[:, 0:1] = p1 / wsum
    w2_ref[:, 1:2] = p2 / wsum

    oh1 = (iota == i1).astype(jnp.float32)
    oh2 = (iota == i2).astype(jnp.float32)
    cnt1 = jnp.sum(oh1, axis=0, keepdims=True)
    cnt2 = jnp.sum(oh2, axis=0, keepdims=True)
    cnt = cnt1 + cnt2
    padded = jnp.floor((cnt + (TTG - 1.0)) * (1.0 / TTG)) * TTG
    r8 = jax.lax.broadcasted_iota(jnp.int32, (E, E), 0)
    c8 = jax.lax.broadcasted_iota(jnp.int32, (E, E), 1)
    u8 = (r8 < c8).astype(jnp.float32)
    start = jax.lax.dot_general(
        padded, u8, (((1,), (0,)), ((), ())),
        preferred_element_type=jnp.float32,
        precision=jax.lax.Precision.HIGHEST,
    )
    off2 = start + cnt1

    CT = 512
    rl = jax.lax.broadcasted_iota(jnp.int32, (CT, CT), 0)
    cl = jax.lax.broadcasted_iota(jnp.int32, (CT, CT), 1)
    ltri = (cl < rl).astype(jnp.float32)
    run1 = jnp.zeros((1, E), jnp.float32)
    run2 = jnp.zeros((1, E), jnp.float32)
    for c in range(T // CT):
        sl = slice(c * CT, (c + 1) * CT)
        o1 = oh1[sl]
        e1 = jax.lax.dot_general(
            ltri, o1, (((1,), (0,)), ((), ())),
            preferred_element_type=jnp.float32,
            precision=jax.lax.Precision.HIGHEST,
        ) + run1
        pos_ref[0, sl] = jnp.sum(o1 * (start + e1), axis=1).astype(jnp.int32)
        run1 = run1 + jnp.sum(o1, axis=0, keepdims=True)
        o2 = oh2[sl]
        e2 = jax.lax.dot_general(
            ltri, o2, (((1,), (0,)), ((), ())),
            preferred_element_type=jnp.float32,
            precision=jax.lax.Precision.HIGHEST,
        ) + run2
        pos_ref[1, sl] = jnp.sum(o2 * (off2 + e2), axis=1).astype(jnp.int32)
        run2 = run2 + jnp.sum(o2, axis=0, keepdims=True)

    gi = jax.lax.broadcasted_iota(jnp.int32, (128, E), 0).astype(jnp.float32) * TTG
    mape = jnp.sum((gi >= start).astype(jnp.int32), axis=1) - 1
    used = jnp.sum(padded)
    mape = jnp.where(gi[:, 0] < used, mape, -1)
    tmap_ref[0, :] = mape


def _dispatch_body(x_hbm, posr_hbm, xs_hbm, xb0, xb1, pb0, pb1,
                   sa0, sa1, sb0, sb1):
    wid = lax.axis_index("s") * 2 + lax.axis_index("c")
    t0 = wid * 64

    pltpu.sync_copy(posr_hbm.at[0, wid], pb0)
    pltpu.sync_copy(posr_hbm.at[1, wid], pb1)

    xbufs = (xb0, xb1)
    semsa = (sa0, sa1)
    semsb = (sb0, sb1)
    scat = [None, None, None, None]
    for j in range(4):
        b = xbufs[j % 2]
        if scat[2 * (j % 2)] is not None:
            scat[2 * (j % 2)].wait()
            scat[2 * (j % 2) + 1].wait()
        pltpu.sync_copy(x_hbm.at[pl.ds(t0 + CH * j, CH)], b)
        scat[2 * (j % 2)] = pltpu.async_copy(
            b, xs_hbm.at[pb0.at[j]], semsa[j % 2])
        scat[2 * (j % 2) + 1] = pltpu.async_copy(
            b, xs_hbm.at[pb1.at[j]], semsb[j % 2])
    for s in scat:
        s.wait()


def _expert_body(m_ref, xs_ref, wg_ref, wu_ref, wd_ref, out_ref):
    g = pl.program_id(0)
    e = m_ref[g]

    @pl.when(e >= 0)
    def _():
        x = xs_ref[...].astype(jnp.bfloat16)
        gg = jax.lax.dot_general(
            x, wg_ref[0].astype(jnp.bfloat16), (((1,), (1,)), ((), ())),
            preferred_element_type=jnp.float32,
        )
        uu = jax.lax.dot_general(
            x, wu_ref[0].astype(jnp.bfloat16), (((1,), (1,)), ((), ())),
            preferred_element_type=jnp.float32,
        )
        h = (gg * jax.nn.sigmoid(gg)) * uu
        o = jax.lax.dot_general(
            h.astype(jnp.bfloat16), wd_ref[0].astype(jnp.bfloat16),
            (((1,), (1,)), ((), ())),
            preferred_element_type=jnp.float32,
        )
        out_ref[...] = o


def _combine_body(os_hbm, pos16_hbm, out_hbm, b0, b1, posb, g0, g1):
    wid = lax.axis_index("s") * 2 + lax.axis_index("c")
    k = wid // 16
    tb = wid % 16

    pltpu.sync_copy(pos16_hbm.at[k, tb], posb)
    bufs = (b0, b1)
    sems = (g0, g1)
    outd = [None, None]
    for j in range(8):
        b = bufs[j % 2]
        if outd[j % 2] is not None:
            outd[j % 2].wait()
        pltpu.sync_copy(os_hbm.at[posb.at[j]], b)
        outd[j % 2] = pltpu.async_copy(
            b, out_hbm.at[pl.ds(k * T + tb * 128 + 16 * j, 16)], sems[j % 2])
    outd[0].wait()
    outd[1].wait()


def _wadd_body(a_ref, b_ref, w_ref, o_ref):
    a = a_ref[...].astype(jnp.float32)
    b = b_ref[...].astype(jnp.float32)
    o_ref[...] = a * w_ref[:, 0:1] + b * w_ref[:, 1:2]


def kernel(hidden_states, gate_w, w_gate, w_up, w_down):
    B, S, _ = hidden_states.shape
    x = hidden_states.reshape(T, D)

    logits, pos2, w2, tmap = pl.pallas_call(
        _router_body,
        grid=(1,),
        in_specs=[
            pl.BlockSpec((T, D), lambda i: (0, 0)),
            pl.BlockSpec((E, D), lambda i: (0, 0)),
        ],
        out_specs=[
            pl.BlockSpec((T, E), lambda i: (0, 0)),
            pl.BlockSpec((K, T), lambda i: (0, 0)),
            pl.BlockSpec((T, K), lambda i: (0, 0)),
            pl.BlockSpec((1, 128), lambda i: (0, 0)),
        ],
        out_shape=[
            jax.ShapeDtypeStruct((T, E), jnp.float32),
            jax.ShapeDtypeStruct((K, T), jnp.int32),
            jax.ShapeDtypeStruct((T, K), jnp.float32),
            jax.ShapeDtypeStruct((1, 128), jnp.int32),
        ],
    )(x, gate_w)

    posr = pos2.reshape(K, 32, 4, CH)
    pos16 = pos2.reshape(K, 16, 8, 16)

    mesh = plsc.VectorSubcoreMesh(core_axis_name="c", subcore_axis_name="s")
    xs = pl.kernel(
        _dispatch_body,
        out_type=jax.ShapeDtypeStruct((P, D), jnp.float32),
        mesh=mesh,
        compiler_params=pltpu.CompilerParams(needs_layout_passes=False),
        scratch_types=[
            pltpu.VMEM((CH, D), jnp.float32),
            pltpu.VMEM((CH, D), jnp.float32),
            pltpu.VMEM((4, CH), jnp.int32),
            pltpu.VMEM((4, CH), jnp.int32),
            pltpu.SemaphoreType.DMA,
            pltpu.SemaphoreType.DMA,
            pltpu.SemaphoreType.DMA,
            pltpu.SemaphoreType.DMA,
        ],
    )(x, posr)

    grid_spec = pltpu.PrefetchScalarGridSpec(
        num_scalar_prefetch=1,
        grid=(G,),
        in_specs=[
            pl.BlockSpec((TTG, D), lambda g, m: (g, 0)),
            pl.BlockSpec((1, FF, D), lambda g, m: (jnp.maximum(m[g], 0), 0, 0)),
            pl.BlockSpec((1, FF, D), lambda g, m: (jnp.maximum(m[g], 0), 0, 0)),
            pl.BlockSpec((1, D, FF), lambda g, m: (jnp.maximum(m[g], 0), 0, 0)),
        ],
        out_specs=pl.BlockSpec((TTG, D), lambda g, m: (g, 0)),
    )
    os = pl.pallas_call(
        _expert_body,
        grid_spec=grid_spec,
        out_shape=jax.ShapeDtypeStruct((P, D), jnp.float32),
        compiler_params=pltpu.CompilerParams(
            dimension_semantics=("arbitrary",),
        ),
    )(tmap[0], xs, w_gate, w_up, w_down)

    out2 = pl.kernel(
        _combine_body,
        out_type=jax.ShapeDtypeStruct((K * T, D), jnp.float32),
        mesh=plsc.VectorSubcoreMesh(core_axis_name="c", subcore_axis_name="s"),
        compiler_params=pltpu.CompilerParams(needs_layout_passes=False),
        scratch_types=[
            pltpu.VMEM((16, D), jnp.float32),
            pltpu.VMEM((16, D), jnp.float32),
            pltpu.VMEM((8, 16), jnp.int32),
            pltpu.SemaphoreType.DMA,
            pltpu.SemaphoreType.DMA,
        ],
    )(os, pos16)

    out = pl.pallas_call(
        _wadd_body,
        grid=(8,),
        in_specs=[
            pl.BlockSpec((256, D), lambda i: (i, 0)),
            pl.BlockSpec((256, D), lambda i: (i + 8, 0)),
            pl.BlockSpec((256, K), lambda i: (i, 0)),
        ],
        out_specs=pl.BlockSpec((256, D), lambda i: (i, 0)),
        out_shape=jax.ShapeDtypeStruct((T, D), jnp.float32),
    )(out2, out2, w2)

    return out.reshape(B, S, D), logits


# speedup vs baseline: 1.7845x; 1.0374x over previous
import functools

import jax
import jax.numpy as jnp
from jax import lax
from jax.experimental import pallas as pl
from jax.experimental.pallas import tpu as pltpu
from jax.experimental.pallas import tpu_sc as plsc

E = 8
D = 2048
FF = 768
T = 2048
K = 2
TTG = 256
P = T * K + E * TTG
G = P // TTG

NW = 32
CH = 16


def _router_body(x_ref, gw_ref, logits_ref, pos_ref, w2_ref, tmap_ref):
    x = x_ref[...]
    gw = gw_ref[...]
    logits = jax.lax.dot_general(
        x, gw, (((1,), (1,)), ((), ())), preferred_element_type=jnp.float32,
    )
    logits_ref[...] = logits
    m = jnp.max(logits, axis=1, keepdims=True)
    ex = jnp.exp(logits - m)
    probs = ex / jnp.sum(ex, axis=1, keepdims=True)
    iota = jax.lax.broadcasted_iota(jnp.int32, probs.shape, 1)
    p1 = jnp.max(probs, axis=1, keepdims=True)
    i1 = jnp.min(jnp.where(probs == p1, iota, E), axis=1, keepdims=True)
    masked = jnp.where(iota == i1, -1.0, probs)
    p2 = jnp.max(masked, axis=1, keepdims=True)
    i2 = jnp.min(jnp.where(masked == p2, iota, E), axis=1, keepdims=True)
    wsum = p1 + p2
    w2_ref[:, 0:1] = p1 / wsum
    w2_ref[:, 1:2] = p2 / wsum

    oh1 = (iota == i1).astype(jnp.float32)
    oh2 = (iota == i2).astype(jnp.float32)
    cnt1 = jnp.sum(oh1, axis=0, keepdims=True)
    cnt2 = jnp.sum(oh2, axis=0, keepdims=True)
    cnt = cnt1 + cnt2
    padded = jnp.floor((cnt + (TTG - 1.0)) * (1.0 / TTG)) * TTG
    r8 = jax.lax.broadcasted_iota(jnp.int32, (E, E), 0)
    c8 = jax.lax.broadcasted_iota(jnp.int32, (E, E), 1)
    u8 = (r8 < c8).astype(jnp.float32)
    start = jax.lax.dot_general(
        padded, u8, (((1,), (0,)), ((), ())),
        preferred_element_type=jnp.float32,
        precision=jax.lax.Precision.HIGHEST,
    )
    off2 = start + cnt1

    CT = 512
    rl = jax.lax.broadcasted_iota(jnp.int32, (CT, CT), 0)
    cl = jax.lax.broadcasted_iota(jnp.int32, (CT, CT), 1)
    ltri = (cl < rl).astype(jnp.float32)
    run1 = jnp.zeros((1, E), jnp.float32)
    run2 = jnp.zeros((1, E), jnp.float32)
    for c in range(T // CT):
        sl = slice(c * CT, (c + 1) * CT)
        o1 = oh1[sl]
        e1 = jax.lax.dot_general(
            ltri, o1, (((1,), (0,)), ((), ())),
            preferred_element_type=jnp.float32,
            precision=jax.lax.Precision.HIGHEST,
        ) + run1
        pos_ref[0, sl] = jnp.sum(o1 * (start + e1), axis=1).astype(jnp.int32)
        run1 = run1 + jnp.sum(o1, axis=0, keepdims=True)
        o2 = oh2[sl]
        e2 = jax.lax.dot_general(
            ltri, o2, (((1,), (0,)), ((), ())),
            preferred_element_type=jnp.float32,
            precision=jax.lax.Precision.HIGHEST,
        ) + run2
        pos_ref[1, sl] = jnp.sum(o2 * (off2 + e2), axis=1).astype(jnp.int32)
        run2 = run2 + jnp.sum(o2, axis=0, keepdims=True)

    gi = jax.lax.broadcasted_iota(jnp.int32, (128, E), 0).astype(jnp.float32) * TTG
    mape = jnp.sum((gi >= start).astype(jnp.int32), axis=1) - 1
    used = jnp.sum(padded)
    valid = gi[:, 0] < used
    tmap_ref[0, :] = jnp.where(valid, mape, -1)
    ti = jax.lax.broadcasted_iota(jnp.int32, (128, E), 0)[:, 0]
    last_tile = jnp.sum(valid.astype(jnp.int32)) - 1
    last_e = jnp.sum(jnp.where(ti == last_tile, mape, 0))
    tmap_ref[1, :] = jnp.where(valid, mape, last_e)
    tmap_ref[2, :] = jnp.minimum(ti, last_tile)


def _dispatch_body(x_hbm, posr_hbm, xs_hbm, xb0, xb1, pb0, pb1,
                   sa0, sa1, sb0, sb1):
    wid = lax.axis_index("s") * 2 + lax.axis_index("c")
    t0 = wid * 64

    pltpu.sync_copy(posr_hbm.at[0, wid], pb0)
    pltpu.sync_copy(posr_hbm.at[1, wid], pb1)

    xbufs = (xb0, xb1)
    semsa = (sa0, sa1)
    semsb = (sb0, sb1)
    scat = [None, None, None, None]
    for j in range(4):
        b = xbufs[j % 2]
        if scat[2 * (j % 2)] is not None:
            scat[2 * (j % 2)].wait()
            scat[2 * (j % 2) + 1].wait()
        pltpu.sync_copy(x_hbm.at[pl.ds(t0 + CH * j, CH)], b)
        scat[2 * (j % 2)] = pltpu.async_copy(
            b, xs_hbm.at[pb0.at[j]], semsa[j % 2])
        scat[2 * (j % 2) + 1] = pltpu.async_copy(
            b, xs_hbm.at[pb1.at[j]], semsb[j % 2])
    for s in scat:
        s.wait()


def _expert_body(m_ref, xs_ref, wg_ref, wu_ref, wd_ref, out_ref):
    g = pl.program_id(0)
    e = m_ref[0, g]

    @pl.when(e >= 0)
    def _():
        x = xs_ref[...].astype(jnp.bfloat16)
        gg = jax.lax.dot_general(
            x, wg_ref[0].astype(jnp.bfloat16), (((1,), (1,)), ((), ())),
            preferred_element_type=jnp.float32,
        )
        uu = jax.lax.dot_general(
            x, wu_ref[0].astype(jnp.bfloat16), (((1,), (1,)), ((), ())),
            preferred_element_type=jnp.float32,
        )
        h = (gg * jax.nn.sigmoid(gg)) * uu
        o = jax.lax.dot_general(
            h.astype(jnp.bfloat16), wd_ref[0].astype(jnp.bfloat16),
            (((1,), (1,)), ((), ())),
            preferred_element_type=jnp.float32,
        )
        out_ref[...] = o


def _combine_body(os_hbm, pos16_hbm, out_hbm, b0, b1, posb, g0, g1):
    wid = lax.axis_index("s") * 2 + lax.axis_index("c")
    k = wid // 16
    tb = wid % 16

    pltpu.sync_copy(pos16_hbm.at[k, tb], posb)
    bufs = (b0, b1)
    sems = (g0, g1)
    outd = [None, None]
    for j in range(8):
        b = bufs[j % 2]
        if outd[j % 2] is not None:
            outd[j % 2].wait()
        pltpu.sync_copy(os_hbm.at[posb.at[j]], b)
        outd[j % 2] = pltpu.async_copy(
            b, out_hbm.at[pl.ds(k * T + tb * 128 + 16 * j, 16)], sems[j % 2])
    outd[0].wait()
    outd[1].wait()


def _wadd_body(a_ref, b_ref, w_ref, o_ref):
    a = a_ref[...].astype(jnp.float32)
    b = b_ref[...].astype(jnp.float32)
    o_ref[...] = a * w_ref[:, 0:1] + b * w_ref[:, 1:2]


def kernel(hidden_states, gate_w, w_gate, w_up, w_down):
    B, S, _ = hidden_states.shape
    x = hidden_states.reshape(T, D)

    logits, pos2, w2, tmap = pl.pallas_call(
        _router_body,
        grid=(1,),
        in_specs=[
            pl.BlockSpec((T, D), lambda i: (0, 0)),
            pl.BlockSpec((E, D), lambda i: (0, 0)),
        ],
        out_specs=[
            pl.BlockSpec((T, E), lambda i: (0, 0)),
            pl.BlockSpec((K, T), lambda i: (0, 0)),
            pl.BlockSpec((T, K), lambda i: (0, 0)),
            pl.BlockSpec((3, 128), lambda i: (0, 0)),
        ],
        out_shape=[
            jax.ShapeDtypeStruct((T, E), jnp.float32),
            jax.ShapeDtypeStruct((K, T), jnp.int32),
            jax.ShapeDtypeStruct((T, K), jnp.float32),
            jax.ShapeDtypeStruct((3, 128), jnp.int32),
        ],
    )(x, gate_w)

    posr = pos2.reshape(K, 32, 4, CH)
    pos16 = pos2.reshape(K, 16, 8, 16)

    mesh = plsc.VectorSubcoreMesh(core_axis_name="c", subcore_axis_name="s")
    xs = pl.kernel(
        _dispatch_body,
        out_type=jax.ShapeDtypeStruct((P, D), jnp.float32),
        mesh=mesh,
        compiler_params=pltpu.CompilerParams(needs_layout_passes=False),
        scratch_types=[
            pltpu.VMEM((CH, D), jnp.float32),
            pltpu.VMEM((CH, D), jnp.float32),
            pltpu.VMEM((4, CH), jnp.int32),
            pltpu.VMEM((4, CH), jnp.int32),
            pltpu.SemaphoreType.DMA,
            pltpu.SemaphoreType.DMA,
            pltpu.SemaphoreType.DMA,
            pltpu.SemaphoreType.DMA,
        ],
    )(x, posr)

    grid_spec = pltpu.PrefetchScalarGridSpec(
        num_scalar_prefetch=1,
        grid=(G,),
        in_specs=[
            pl.BlockSpec((TTG, D), lambda g, m: (m[2, g], 0)),
            pl.BlockSpec((1, FF, D), lambda g, m: (m[1, g], 0, 0)),
            pl.BlockSpec((1, FF, D), lambda g, m: (m[1, g], 0, 0)),
            pl.BlockSpec((1, D, FF), lambda g, m: (m[1, g], 0, 0)),
        ],
        out_specs=pl.BlockSpec((TTG, D), lambda g, m: (m[2, g], 0)),
    )
    os = pl.pallas_call(
        _expert_body,
        grid_spec=grid_spec,
        out_shape=jax.ShapeDtypeStruct((P, D), jnp.float32),
        compiler_params=pltpu.CompilerParams(
            dimension_semantics=("arbitrary",),
        ),
    )(tmap, xs, w_gate, w_up, w_down)

    out2 = pl.kernel(
        _combine_body,
        out_type=jax.ShapeDtypeStruct((K * T, D), jnp.float32),
        mesh=plsc.VectorSubcoreMesh(core_axis_name="c", subcore_axis_name="s"),
        compiler_params=pltpu.CompilerParams(needs_layout_passes=False),
        scratch_types=[
            pltpu.VMEM((16, D), jnp.float32),
            pltpu.VMEM((16, D), jnp.float32),
            pltpu.VMEM((8, 16), jnp.int32),
            pltpu.SemaphoreType.DMA,
            pltpu.SemaphoreType.DMA,
        ],
    )(os, pos16)

    out = pl.pallas_call(
        _wadd_body,
        grid=(8,),
        in_specs=[
            pl.BlockSpec((256, D), lambda i: (i, 0)),
            pl.BlockSpec((256, D), lambda i: (i + 8, 0)),
            pl.BlockSpec((256, K), lambda i: (i, 0)),
        ],
        out_specs=pl.BlockSpec((256, D), lambda i: (i, 0)),
        out_shape=jax.ShapeDtypeStruct((T, D), jnp.float32),
    )(out2, out2, w2)

    return out.reshape(B, S, D), logits


# speedup vs baseline: 1.8027x; 1.0102x over previous
import functools

import jax
import jax.numpy as jnp
from jax import lax
from jax.experimental import pallas as pl
from jax.experimental.pallas import tpu as pltpu
from jax.experimental.pallas import tpu_sc as plsc

E = 8
D = 2048
FF = 768
T = 2048
K = 2
TTG = 256
P = T * K + E * TTG
G = P // TTG

NW = 32
CH = 16


def _router_body(x_ref, gw_ref, logits_ref, pos_ref, w2_ref, tmap_ref):
    x = x_ref[...]
    gw = gw_ref[...]
    logits = jax.lax.dot_general(
        x, gw, (((1,), (1,)), ((), ())), preferred_element_type=jnp.float32,
    )
    logits_ref[...] = logits
    m = jnp.max(logits, axis=1, keepdims=True)
    ex = jnp.exp(logits - m)
    probs = ex / jnp.sum(ex, axis=1, keepdims=True)
    iota = jax.lax.broadcasted_iota(jnp.int32, probs.shape, 1)
    p1 = jnp.max(probs, axis=1, keepdims=True)
    i1 = jnp.min(jnp.where(probs == p1, iota, E), axis=1, keepdims=True)
    masked = jnp.where(iota == i1, -1.0, probs)
    p2 = jnp.max(masked, axis=1, keepdims=True)
    i2 = jnp.min(jnp.where(masked == p2, iota, E), axis=1, keepdims=True)
    wsum = p1 + p2
    w2_ref[:, 0:1] = p1 / wsum
    w2_ref[:, 1:2] = p2 / wsum

    oh1 = (iota == i1).astype(jnp.float32)
    oh2 = (iota == i2).astype(jnp.float32)
    cnt1 = jnp.sum(oh1, axis=0, keepdims=True)
    cnt2 = jnp.sum(oh2, axis=0, keepdims=True)
    cnt = cnt1 + cnt2
    padded = jnp.floor((cnt + (TTG - 1.0)) * (1.0 / TTG)) * TTG
    r8 = jax.lax.broadcasted_iota(jnp.int32, (E, E), 0)
    c8 = jax.lax.broadcasted_iota(jnp.int32, (E, E), 1)
    u8 = (r8 < c8).astype(jnp.float32)
    start = jax.lax.dot_general(
        padded, u8, (((1,), (0,)), ((), ())),
        preferred_element_type=jnp.float32,
        precision=jax.lax.Precision.HIGHEST,
    )
    off2 = start + cnt1

    CT = 512
    rl = jax.lax.broadcasted_iota(jnp.int32, (CT, CT), 0)
    cl = jax.lax.broadcasted_iota(jnp.int32, (CT, CT), 1)
    ltri = (cl < rl).astype(jnp.float32)
    run1 = jnp.zeros((1, E), jnp.float32)
    run2 = jnp.zeros((1, E), jnp.float32)
    for c in range(T // CT):
        sl = slice(c * CT, (c + 1) * CT)
        o1 = oh1[sl]
        e1 = jax.lax.dot_general(
            ltri, o1, (((1,), (0,)), ((), ())),
            preferred_element_type=jnp.float32,
            precision=jax.lax.Precision.HIGHEST,
        ) + run1
        pos_ref[0, sl] = jnp.sum(o1 * (start + e1), axis=1).astype(jnp.int32)
        run1 = run1 + jnp.sum(o1, axis=0, keepdims=True)
        o2 = oh2[sl]
        e2 = jax.lax.dot_general(
            ltri, o2, (((1,), (0,)), ((), ())),
            preferred_element_type=jnp.float32,
            precision=jax.lax.Precision.HIGHEST,
        ) + run2
        pos_ref[1, sl] = jnp.sum(o2 * (off2 + e2), axis=1).astype(jnp.int32)
        run2 = run2 + jnp.sum(o2, axis=0, keepdims=True)

    gi = jax.lax.broadcasted_iota(jnp.int32, (128, E), 0).astype(jnp.float32) * TTG
    mape = jnp.sum((gi >= start).astype(jnp.int32), axis=1) - 1
    used = jnp.sum(padded)
    valid = gi[:, 0] < used
    tmap_ref[0, :] = jnp.where(valid, mape, -1)
    ti = jax.lax.broadcasted_iota(jnp.int32, (128, E), 0)[:, 0]
    last_tile = jnp.sum(valid.astype(jnp.int32)) - 1
    last_e = jnp.sum(jnp.where(ti == last_tile, mape, 0))
    tmap_ref[1, :] = jnp.where(valid, mape, last_e)
    tmap_ref[2, :] = jnp.minimum(ti, last_tile)


def _dispatch_body(x_hbm, pos_hbm, xs_hbm, xb0, xb1, pb0, pb1,
                   sa0, sa1, sb0, sb1):
    wid = lax.axis_index("s") * 2 + lax.axis_index("c")
    t0 = wid * 64

    pltpu.sync_copy(pos_hbm.at[0, pl.ds(t0, 64)], pb0)
    pltpu.sync_copy(pos_hbm.at[1, pl.ds(t0, 64)], pb1)

    xbufs = (xb0, xb1)
    semsa = (sa0, sa1)
    semsb = (sb0, sb1)
    scat = [None, None, None, None]
    for j in range(4):
        b = xbufs[j % 2]
        if scat[2 * (j % 2)] is not None:
            scat[2 * (j % 2)].wait()
            scat[2 * (j % 2) + 1].wait()
        pltpu.sync_copy(x_hbm.at[pl.ds(t0 + CH * j, CH)], b)
        scat[2 * (j % 2)] = pltpu.async_copy(
            b, xs_hbm.at[pb0.at[pl.ds(CH * j, CH)]], semsa[j % 2])
        scat[2 * (j % 2) + 1] = pltpu.async_copy(
            b, xs_hbm.at[pb1.at[pl.ds(CH * j, CH)]], semsb[j % 2])
    for s in scat:
        s.wait()


def _expert_body(m_ref, xs_ref, wg_ref, wu_ref, wd_ref, out_ref):
    g = pl.program_id(0)
    e = m_ref[0, g]

    @pl.when(e >= 0)
    def _():
        x = xs_ref[...].astype(jnp.bfloat16)
        gg = jax.lax.dot_general(
            x, wg_ref[0].astype(jnp.bfloat16), (((1,), (1,)), ((), ())),
            preferred_element_type=jnp.float32,
        )
        uu = jax.lax.dot_general(
            x, wu_ref[0].astype(jnp.bfloat16), (((1,), (1,)), ((), ())),
            preferred_element_type=jnp.float32,
        )
        h = (gg * jax.nn.sigmoid(gg)) * uu
        o = jax.lax.dot_general(
            h.astype(jnp.bfloat16), wd_ref[0].astype(jnp.bfloat16),
            (((1,), (1,)), ((), ())),
            preferred_element_type=jnp.float32,
        )
        out_ref[...] = o


def _combine_body(os_hbm, pos_hbm, out_hbm, b0, b1, posb, g0, g1):
    wid = lax.axis_index("s") * 2 + lax.axis_index("c")
    k = wid // 16
    tb = wid % 16

    pltpu.sync_copy(pos_hbm.at[k, pl.ds(tb * 128, 128)], posb)
    bufs = (b0, b1)
    sems = (g0, g1)
    outd = [None, None]
    for j in range(8):
        b = bufs[j % 2]
        if outd[j % 2] is not None:
            outd[j % 2].wait()
        pltpu.sync_copy(os_hbm.at[posb.at[pl.ds(16 * j, 16)]], b)
        outd[j % 2] = pltpu.async_copy(
            b, out_hbm.at[pl.ds(k * T + tb * 128 + 16 * j, 16)], sems[j % 2])
    outd[0].wait()
    outd[1].wait()


def _wadd_body(a_ref, b_ref, w_ref, o_ref):
    a = a_ref[...].astype(jnp.float32)
    b = b_ref[...].astype(jnp.float32)
    o_ref[...] = a * w_ref[:, 0:1] + b * w_ref[:, 1:2]


def kernel(hidden_states, gate_w, w_gate, w_up, w_down):
    B, S, _ = hidden_states.shape
    x = hidden_states.reshape(T, D)

    logits, pos2, w2, tmap = pl.pallas_call(
        _router_body,
        grid=(1,),
        in_specs=[
            pl.BlockSpec((T, D), lambda i: (0, 0)),
            pl.BlockSpec((E, D), lambda i: (0, 0)),
        ],
        out_specs=[
            pl.BlockSpec((T, E), lambda i: (0, 0)),
            pl.BlockSpec((K, T), lambda i: (0, 0)),
            pl.BlockSpec((T, K), lambda i: (0, 0)),
            pl.BlockSpec((3, 128), lambda i: (0, 0)),
        ],
        out_shape=[
            jax.ShapeDtypeStruct((T, E), jnp.float32),
            jax.ShapeDtypeStruct((K, T), jnp.int32),
            jax.ShapeDtypeStruct((T, K), jnp.float32),
            jax.ShapeDtypeStruct((3, 128), jnp.int32),
        ],
    )(x, gate_w)

    mesh = plsc.VectorSubcoreMesh(core_axis_name="c", subcore_axis_name="s")
    xs = pl.kernel(
        _dispatch_body,
        out_type=jax.ShapeDtypeStruct((P, D), jnp.float32),
        mesh=mesh,
        compiler_params=pltpu.CompilerParams(needs_layout_passes=False),
        scratch_types=[
            pltpu.VMEM((CH, D), jnp.float32),
            pltpu.VMEM((CH, D), jnp.float32),
            pltpu.VMEM((64,), jnp.int32),
            pltpu.VMEM((64,), jnp.int32),
            pltpu.SemaphoreType.DMA,
            pltpu.SemaphoreType.DMA,
            pltpu.SemaphoreType.DMA,
            pltpu.SemaphoreType.DMA,
        ],
    )(x, pos2)

    grid_spec = pltpu.PrefetchScalarGridSpec(
        num_scalar_prefetch=1,
        grid=(G,),
        in_specs=[
            pl.BlockSpec((TTG, D), lambda g, m: (m[2, g], 0)),
            pl.BlockSpec((1, FF, D), lambda g, m: (m[1, g], 0, 0)),
            pl.BlockSpec((1, FF, D), lambda g, m: (m[1, g], 0, 0)),
            pl.BlockSpec((1, D, FF), lambda g, m: (m[1, g], 0, 0)),
        ],
        out_specs=pl.BlockSpec((TTG, D), lambda g, m: (m[2, g], 0)),
    )
    os = pl.pallas_call(
        _expert_body,
        grid_spec=grid_spec,
        out_shape=jax.ShapeDtypeStruct((P, D), jnp.float32),
        compiler_params=pltpu.CompilerParams(
            dimension_semantics=("arbitrary",),
        ),
    )(tmap, xs, w_gate, w_up, w_down)

    out2 = pl.kernel(
        _combine_body,
        out_type=jax.ShapeDtypeStruct((K * T, D), jnp.float32),
        mesh=plsc.VectorSubcoreMesh(core_axis_name="c", subcore_axis_name="s"),
        compiler_params=pltpu.CompilerParams(needs_layout_passes=False),
        scratch_types=[
            pltpu.VMEM((16, D), jnp.float32),
            pltpu.VMEM((16, D), jnp.float32),
            pltpu.VMEM((128,), jnp.int32),
            pltpu.SemaphoreType.DMA,
            pltpu.SemaphoreType.DMA,
        ],
    )(os, pos2)

    out = pl.pallas_call(
        _wadd_body,
        grid=(8,),
        in_specs=[
            pl.BlockSpec((256, D), lambda i: (i, 0)),
            pl.BlockSpec((256, D), lambda i: (i + 8, 0)),
            pl.BlockSpec((256, K), lambda i: (i, 0)),
        ],
        out_specs=pl.BlockSpec((256, D), lambda i: (i, 0)),
        out_shape=jax.ShapeDtypeStruct((T, D), jnp.float32),
    )(out2, out2, w2)

    return out.reshape(B, S, D), logits


# speedup vs baseline: 1.8126x; 1.0055x over previous
import functools

import jax
import jax.numpy as jnp
from jax import lax
from jax.experimental import pallas as pl
from jax.experimental.pallas import tpu as pltpu
from jax.experimental.pallas import tpu_sc as plsc

E = 8
D = 2048
FF = 768
T = 2048
K = 2
TTG = 256
P = T * K + E * TTG
G = P // TTG

NW = 32
CH = 16


def _router_body(x_ref, gw_ref, logits_ref, pos_ref, w2_ref, tmap_ref):
    x = x_ref[...]
    gw = gw_ref[...]
    logits = jax.lax.dot_general(
        x, gw, (((1,), (1,)), ((), ())), preferred_element_type=jnp.float32,
    )
    logits_ref[...] = logits
    m = jnp.max(logits, axis=1, keepdims=True)
    ex = jnp.exp(logits - m)
    probs = ex / jnp.sum(ex, axis=1, keepdims=True)
    iota = jax.lax.broadcasted_iota(jnp.int32, probs.shape, 1)
    p1 = jnp.max(probs, axis=1, keepdims=True)
    i1 = jnp.min(jnp.where(probs == p1, iota, E), axis=1, keepdims=True)
    masked = jnp.where(iota == i1, -1.0, probs)
    p2 = jnp.max(masked, axis=1, keepdims=True)
    i2 = jnp.min(jnp.where(masked == p2, iota, E), axis=1, keepdims=True)
    wsum = p1 + p2
    w2_ref[:, 0:1] = p1 / wsum
    w2_ref[:, 1:2] = p2 / wsum

    oh1 = (iota == i1).astype(jnp.float32)
    oh2 = (iota == i2).astype(jnp.float32)
    cnt1 = jnp.sum(oh1, axis=0, keepdims=True)
    cnt2 = jnp.sum(oh2, axis=0, keepdims=True)
    cnt = cnt1 + cnt2
    padded = jnp.floor((cnt + (TTG - 1.0)) * (1.0 / TTG)) * TTG
    r8 = jax.lax.broadcasted_iota(jnp.int32, (E, E), 0)
    c8 = jax.lax.broadcasted_iota(jnp.int32, (E, E), 1)
    u8 = (r8 < c8).astype(jnp.float32)
    start = jax.lax.dot_general(
        padded, u8, (((1,), (0,)), ((), ())),
        preferred_element_type=jnp.float32,
        precision=jax.lax.Precision.HIGHEST,
    )
    off2 = start + cnt1

    CT = 512
    rl = jax.lax.broadcasted_iota(jnp.int32, (CT, CT), 0)
    cl = jax.lax.broadcasted_iota(jnp.int32, (CT, CT), 1)
    ltri = (cl < rl).astype(jnp.float32)
    run1 = jnp.zeros((1, E), jnp.float32)
    run2 = jnp.zeros((1, E), jnp.float32)
    for c in range(T // CT):
        sl = slice(c * CT, (c + 1) * CT)
        o1 = oh1[sl]
        e1 = jax.lax.dot_general(
            ltri, o1, (((1,), (0,)), ((), ())),
            preferred_element_type=jnp.float32,
            precision=jax.lax.Precision.HIGHEST,
        ) + run1
        pos_ref[0, sl] = jnp.sum(o1 * (start + e1), axis=1).astype(jnp.int32)
        run1 = run1 + jnp.sum(o1, axis=0, keepdims=True)
        o2 = oh2[sl]
        e2 = jax.lax.dot_general(
            ltri, o2, (((1,), (0,)), ((), ())),
            preferred_element_type=jnp.float32,
            precision=jax.lax.Precision.HIGHEST,
        ) + run2
        pos_ref[1, sl] = jnp.sum(o2 * (off2 + e2), axis=1).astype(jnp.int32)
        run2 = run2 + jnp.sum(o2, axis=0, keepdims=True)

    gi = jax.lax.broadcasted_iota(jnp.int32, (128, E), 0).astype(jnp.float32) * TTG
    mape = jnp.sum((gi >= start).astype(jnp.int32), axis=1) - 1
    used = jnp.sum(padded)
    valid = gi[:, 0] < used
    tmap_ref[0, :] = jnp.where(valid, mape, -1)
    ti = jax.lax.broadcasted_iota(jnp.int32, (128, E), 0)[:, 0]
    last_tile = jnp.sum(valid.astype(jnp.int32)) - 1
    last_e = jnp.sum(jnp.where(ti == last_tile, mape, 0))
    tmap_ref[1, :] = jnp.where(valid, mape, last_e)
    tmap_ref[2, :] = jnp.minimum(ti, last_tile)


def _dispatch_body(x_hbm, pos_hbm, xs_hbm, xb0, xb1, pb0, pb1,
                   sa0, sa1, sb0, sb1):
    wid = lax.axis_index("s") * 2 + lax.axis_index("c")
    t0 = wid * 64

    pltpu.sync_copy(pos_hbm.at[0, pl.ds(t0, 64)], pb0)
    pltpu.sync_copy(pos_hbm.at[1, pl.ds(t0, 64)], pb1)

    xbufs = (xb0, xb1)
    semsa = (sa0, sa1)
    semsb = (sb0, sb1)
    scat = [None, None, None, None]
    for j in range(4):
        b = xbufs[j % 2]
        if scat[2 * (j % 2)] is not None:
            scat[2 * (j % 2)].wait()
            scat[2 * (j % 2) + 1].wait()
        pltpu.sync_copy(x_hbm.at[pl.ds(t0 + CH * j, CH)], b)
        scat[2 * (j % 2)] = pltpu.async_copy(
            b, xs_hbm.at[pb0.at[pl.ds(CH * j, CH)]], semsa[j % 2])
        scat[2 * (j % 2) + 1] = pltpu.async_copy(
            b, xs_hbm.at[pb1.at[pl.ds(CH * j, CH)]], semsb[j % 2])
    for s in scat:
        s.wait()


def _expert_body(m_ref, xs_ref, wg_ref, wu_ref, wd_ref, out_ref):
    g = pl.program_id(0)
    e = m_ref[0, g]

    @pl.when(e >= 0)
    def _():
        x = xs_ref[...].astype(jnp.bfloat16)
        gg = jax.lax.dot_general(
            x, wg_ref[0].astype(jnp.bfloat16), (((1,), (1,)), ((), ())),
            preferred_element_type=jnp.float32,
        )
        uu = jax.lax.dot_general(
            x, wu_ref[0].astype(jnp.bfloat16), (((1,), (1,)), ((), ())),
            preferred_element_type=jnp.float32,
        )
        h = (gg * jax.nn.sigmoid(gg)) * uu
        o = jax.lax.dot_general(
            h.astype(jnp.bfloat16), wd_ref[0].astype(jnp.bfloat16),
            (((1,), (1,)), ((), ())),
            preferred_element_type=jnp.float32,
        )
        out_ref[...] = o


def _combine_body(os_hbm, pos_hbm, out_hbm, b0, b1, posb, g0, g1):
    wid = lax.axis_index("s") * 2 + lax.axis_index("c")
    k = wid // 16
    tb = wid % 16

    pltpu.sync_copy(pos_hbm.at[k, pl.ds(tb * 128, 128)], posb)
    bufs = (b0, b1)
    sems = (g0, g1)
    outd = [None, None]
    for j in range(8):
        b = bufs[j % 2]
        if outd[j % 2] is not None:
            outd[j % 2].wait()
        pltpu.sync_copy(os_hbm.at[posb.at[pl.ds(16 * j, 16)]], b)
        outd[j % 2] = pltpu.async_copy(
            b, out_hbm.at[pl.ds(k * T + tb * 128 + 16 * j, 16)], sems[j % 2])
    outd[0].wait()
    outd[1].wait()


def _wadd_body(a_ref, b_ref, w_ref, o_ref):
    a = a_ref[...].astype(jnp.float32)
    b = b_ref[...].astype(jnp.float32)
    o_ref[...] = a * w_ref[:, 0:1] + b * w_ref[:, 1:2]


def kernel(hidden_states, gate_w, w_gate, w_up, w_down):
    B, S, _ = hidden_states.shape
    x = hidden_states.reshape(T, D)

    logits, pos2, w2, tmap = pl.pallas_call(
        _router_body,
        grid=(1,),
        in_specs=[
            pl.BlockSpec((T, D), lambda i: (0, 0)),
            pl.BlockSpec((E, D), lambda i: (0, 0)),
        ],
        out_specs=[
            pl.BlockSpec((T, E), lambda i: (0, 0)),
            pl.BlockSpec((K, T), lambda i: (0, 0)),
            pl.BlockSpec((T, K), lambda i: (0, 0)),
            pl.BlockSpec((3, 128), lambda i: (0, 0)),
        ],
        out_shape=[
            jax.ShapeDtypeStruct((T, E), jnp.float32),
            jax.ShapeDtypeStruct((K, T), jnp.int32),
            jax.ShapeDtypeStruct((T, K), jnp.float32),
            jax.ShapeDtypeStruct((3, 128), jnp.int32),
        ],
    )(x, gate_w)

    mesh = plsc.VectorSubcoreMesh(core_axis_name="c", subcore_axis_name="s")
    xs = pl.kernel(
        _dispatch_body,
        out_type=jax.ShapeDtypeStruct((P, D), jnp.float32),
        mesh=mesh,
        compiler_params=pltpu.CompilerParams(needs_layout_passes=False),
        scratch_types=[
            pltpu.VMEM((CH, D), jnp.float32),
            pltpu.VMEM((CH, D), jnp.float32),
            pltpu.VMEM((64,), jnp.int32),
            pltpu.VMEM((64,), jnp.int32),
            pltpu.SemaphoreType.DMA,
            pltpu.SemaphoreType.DMA,
            pltpu.SemaphoreType.DMA,
            pltpu.SemaphoreType.DMA,
        ],
    )(x, pos2)

    grid_spec = pltpu.PrefetchScalarGridSpec(
        num_scalar_prefetch=1,
        grid=(G,),
        in_specs=[
            pl.BlockSpec((TTG, D), lambda g, m: (m[2, g], 0)),
            pl.BlockSpec((1, FF, D), lambda g, m: (m[1, g], 0, 0)),
            pl.BlockSpec((1, FF, D), lambda g, m: (m[1, g], 0, 0)),
            pl.BlockSpec((1, D, FF), lambda g, m: (m[1, g], 0, 0)),
        ],
        out_specs=pl.BlockSpec((TTG, D), lambda g, m: (m[2, g], 0)),
    )
    os = pl.pallas_call(
        _expert_body,
        grid_spec=grid_spec,
        out_shape=jax.ShapeDtypeStruct((P, D), jnp.float32),
        compiler_params=pltpu.CompilerParams(
            dimension_semantics=("arbitrary",),
        ),
    )(tmap, xs, w_gate, w_up, w_down)

    out2 = pl.kernel(
        _combine_body,
        out_type=jax.ShapeDtypeStruct((K * T, D), jnp.float32),
        mesh=plsc.VectorSubcoreMesh(core_axis_name="c", subcore_axis_name="s"),
        compiler_params=pltpu.CompilerParams(needs_layout_passes=False),
        scratch_types=[
            pltpu.VMEM((16, D), jnp.float32),
            pltpu.VMEM((16, D), jnp.float32),
            pltpu.VMEM((128,), jnp.int32),
            pltpu.SemaphoreType.DMA,
            pltpu.SemaphoreType.DMA,
        ],
    )(os, pos2)

    out = pl.pallas_call(
        _wadd_body,
        grid=(4,),
        in_specs=[
            pl.BlockSpec((512, D), lambda i: (i, 0)),
            pl.BlockSpec((512, D), lambda i: (i + 4, 0)),
            pl.BlockSpec((512, K), lambda i: (i, 0)),
        ],
        out_specs=pl.BlockSpec((512, D), lambda i: (i, 0)),
        out_shape=jax.ShapeDtypeStruct((T, D), jnp.float32),
    )(out2, out2, w2)

    return out.reshape(B, S, D), logits
